# baseline (speedup 1.0000x reference)
import functools

import jax
import jax.numpy as jnp
from jax import lax
from jax.experimental import pallas as pl
from jax.experimental.pallas import tpu as pltpu
from jax.experimental.pallas import tpu_sc as plsc

N_NODES = 10000
N_EDGES = 320000
NODE_DIM = 128
EDGE_DIM = 16
HIDDEN = 128

_NC = 2
_NS = 16
_NW = _NC * _NS

_G_CHUNK = 128
_G_NCHUNKS = 2560
_E_PAD = _G_NCHUNKS * _G_CHUNK
_G_W0 = 192
_G_W1 = 128
_G_NBUF = 4

_S_CHUNK = 80
_S_CHUNKS_PER_W = 125


def _dot(a, b, prec=jax.lax.Precision.HIGHEST):
    return jax.lax.dot_general(a, b, (((1,), (0,)), ((), ())),
                               precision=prec, preferred_element_type=jnp.float32)


def _k1_body(nf_ref, w_ref, w1_ref, nw0b_ref, eb1_ref, outs_ref, outr_ref,
             w1cat_ref, b1n_ref):
    p = _dot(nf_ref[...], w_ref[...])
    outs_ref[...] = p[:, :HIDDEN]
    outr_ref[...] = p[:, HIDDEN:]
    w1n = _dot(w1_ref[...], nw0b_ref[...])
    w1cat_ref[...] = jnp.concatenate([w1_ref[...], w1n], axis=1)
    b1n_ref[...] = _dot(eb1_ref[...], nw0b_ref[...])


def _node_project(nf, w_sr, w1, nw0b, eb1row, interpret=False):
    blk = 2000
    grid = (N_NODES // blk,)
    full = lambda i: (0, 0)
    return pl.pallas_call(
        _k1_body,
        grid=grid,
        in_specs=[pl.BlockSpec((blk, NODE_DIM), lambda i: (i, 0)),
                  pl.BlockSpec((NODE_DIM, 2 * HIDDEN), full),
                  pl.BlockSpec((HIDDEN, EDGE_DIM), full),
                  pl.BlockSpec((EDGE_DIM, HIDDEN), full),
                  pl.BlockSpec((1, EDGE_DIM), full)],
        out_specs=[pl.BlockSpec((blk, HIDDEN), lambda i: (i, 0)),
                   pl.BlockSpec((blk, HIDDEN), lambda i: (i, 0)),
                   pl.BlockSpec((HIDDEN, EDGE_DIM + HIDDEN), full),
                   pl.BlockSpec((1, HIDDEN), full)],
        out_shape=[jax.ShapeDtypeStruct((N_NODES, HIDDEN), jnp.float32),
                   jax.ShapeDtypeStruct((N_NODES, HIDDEN), jnp.float32),
                   jax.ShapeDtypeStruct((HIDDEN, EDGE_DIM + HIDDEN), jnp.float32),
                   jax.ShapeDtypeStruct((1, HIDDEN), jnp.float32)],
        interpret=interpret,
    )(nf, w_sr, w1, nw0b, eb1row)


def _sc_gather(ps, pr, idx_all):
    mesh = plsc.VectorSubcoreMesh(core_axis_name="c", subcore_axis_name="s")

    @functools.partial(
        pl.kernel,
        out_type=jax.ShapeDtypeStruct((2 * _E_PAD, HIDDEN), jnp.float32),
        mesh=mesh,
        scratch_types=[
            pltpu.VMEM((max(_G_W0, _G_W1), _G_CHUNK), jnp.int32),
            pltpu.VMEM((_G_CHUNK, HIDDEN), jnp.float32),
            pltpu.VMEM((_G_CHUNK, HIDDEN), jnp.float32),
            pltpu.VMEM((_G_CHUNK, HIDDEN), jnp.float32),
            pltpu.VMEM((_G_CHUNK, HIDDEN), jnp.float32),
            pltpu.SemaphoreType.DMA,
            pltpu.SemaphoreType.DMA,
            pltpu.SemaphoreType.DMA,
            pltpu.SemaphoreType.DMA,
            pltpu.SemaphoreType.DMA,
            pltpu.SemaphoreType.DMA,
            pltpu.SemaphoreType.DMA,
            pltpu.SemaphoreType.DMA,
        ],
    )
    def k(ps_hbm, pr_hbm, idx_hbm, h_hbm,
          iv, b0, b1, b2, b3, g0, g1, g2, g3, w0, w1, w2, w3):
        cid = lax.axis_index("c")
        sid = lax.axis_index("s")
        n_w = jnp.where(cid == 0, _G_W0, _G_W1)
        start = jnp.where(cid == 0, sid * _G_W0, 16 * _G_W0 + sid * _G_W1)

        @pl.when(cid == 0)
        def _():
            pltpu.sync_copy(idx_hbm.at[pl.ds(sid * _G_W0, _G_W0)],
                            iv.at[pl.ds(0, _G_W0)])

        @pl.when(cid == 1)
        def _():
            pltpu.sync_copy(
                idx_hbm.at[pl.ds(16 * _G_W0 + sid * _G_W1, _G_W1)],
                iv.at[pl.ds(0, _G_W1)])

        bufs = (b0, b1, b2, b3)
        gsem = (g0, g1, g2, g3)
        wsem = (w0, w1, w2, w3)

        def fire_gather(jl, buf, sem):
            cg = start + jl

            @pl.when(cg < _G_NCHUNKS)
            def _():
                pltpu.async_copy(ps_hbm.at[iv.at[jl]], buf, sem)

            @pl.when(cg >= _G_NCHUNKS)
            def _():
                pltpu.async_copy(pr_hbm.at[iv.at[jl]], buf, sem)

        for kk in range(_G_NBUF):
            fire_gather(kk, bufs[kk], gsem[kk])

        def body(t, _):
            jbase = _G_NBUF * t
            for kk in range(_G_NBUF):
                jl = jbase + kk
                cg = start + jl
                pltpu.make_async_copy(ps_hbm.at[iv.at[jl]], bufs[kk],
                                      gsem[kk]).wait()
                pltpu.async_copy(
                    bufs[kk], h_hbm.at[pl.ds(cg * _G_CHUNK, _G_CHUNK)],
                    wsem[kk])
            for kk in range(_G_NBUF):
                jl = jbase + kk

                @pl.when(jl + _G_NBUF < n_w)
                def _():
                    pltpu.make_async_copy(
                        bufs[kk], h_hbm.at[pl.ds(0, _G_CHUNK)],
                        wsem[kk]).wait()
                    fire_gather(jl + _G_NBUF, bufs[kk], gsem[kk])

            return 0

        lax.fori_loop(0, n_w // _G_NBUF, body, 0)
        for kk in range(_G_NBUF):
            pltpu.make_async_copy(bufs[kk], h_hbm.at[pl.ds(0, _G_CHUNK)],
                                  wsem[kk]).wait()

    return k(ps, pr, idx_all)


def _dot3(a, b):
    ah = a.astype(jnp.bfloat16)
    al = (a - ah.astype(jnp.float32)).astype(jnp.bfloat16)
    bh = b.astype(jnp.bfloat16)
    bl = (b - bh.astype(jnp.float32)).astype(jnp.bfloat16)
    d = lambda x, y: jax.lax.dot_general(
        x, y, (((1,), (0,)), ((), ())), preferred_element_type=jnp.float32)
    return d(ah, bh) + d(ah, bl) + d(al, bh)


def _k3_body(hs_ref, hr_ref, ef_ref, we2_ref, b0_ref, g_ref, bt_ref,
             w1cat_ref, b1_ref, b1n_ref, out_ref, q_ref):
    e = ef_ref[...]
    s = _dot3(e, we2_ref[...])
    h = hs_ref[...] + hr_ref[...] + s[:, :HIDDEN] + b0_ref[...]
    mu = jnp.mean(h, axis=-1, keepdims=True)
    d = h - mu
    var = jnp.mean(d * d, axis=-1, keepdims=True)
    h = d / jnp.sqrt(var + 1e-5) * g_ref[...] + bt_ref[...]
    h = h * jax.nn.sigmoid(h)
    r = _dot3(h, w1cat_ref[...])
    out_ref[...] = e + r[:, :EDGE_DIM] + b1_ref[...]
    q_ref[...] = s[:, HIDDEN:] + r[:, EDGE_DIM:] + b1n_ref[...]


def _edge_mlp(hcat, ef, we2, b0, g, bt, w1cat, b1, b1n, interpret=False):
    blk = 1280
    grid = (N_EDGES // blk,)
    off = _E_PAD // blk
    full = lambda i: (0, 0)
    return pl.pallas_call(
        _k3_body,
        grid=grid,
        in_specs=[pl.BlockSpec((blk, HIDDEN), lambda i: (i, 0)),
                  pl.BlockSpec((blk, HIDDEN), lambda i: (i + off, 0)),
                  pl.BlockSpec((blk, EDGE_DIM), lambda i: (i, 0)),
                  pl.BlockSpec((EDGE_DIM, HIDDEN + HIDDEN), full),
                  pl.BlockSpec((1, HIDDEN), full),
                  pl.BlockSpec((1, HIDDEN), full),
                  pl.BlockSpec((1, HIDDEN), full),
                  pl.BlockSpec((HIDDEN, EDGE_DIM + HIDDEN), full),
                  pl.BlockSpec((1, EDGE_DIM), full),
                  pl.BlockSpec((1, HIDDEN), full)],
        out_specs=[pl.BlockSpec((blk, EDGE_DIM), lambda i: (i, 0)),
                   pl.BlockSpec((blk, HIDDEN), lambda i: (i, 0))],
        out_shape=[jax.ShapeDtypeStruct((N_EDGES, EDGE_DIM), jnp.float32),
                   jax.ShapeDtypeStruct((N_EDGES, HIDDEN), jnp.float32)],
        interpret=interpret,
    )(hcat, hcat, ef, we2, b0, g, bt, w1cat, b1, b1n)


def _sc_scatter(eout, ridx2, zeros_tab):
    mesh = plsc.VectorSubcoreMesh(core_axis_name="c", subcore_axis_name="s")

    @functools.partial(
        pl.kernel,
        out_type=jax.ShapeDtypeStruct((_NC, N_NODES, HIDDEN), jnp.float32),
        mesh=mesh,
        scratch_types=[
            pltpu.VMEM((_S_CHUNKS_PER_W, _S_CHUNK), jnp.int32),
            pltpu.VMEM((_S_CHUNK, HIDDEN), jnp.float32),
            pltpu.VMEM((_S_CHUNK, HIDDEN), jnp.float32),
            pltpu.VMEM_SHARED((N_NODES, HIDDEN), jnp.float32),
            pltpu.SemaphoreType.DMA,
            pltpu.SemaphoreType.DMA,
        ],
    )
    def k(eout_hbm, ridx_hbm, zero_hbm, out_hbm, iv, eva, evb, acc, la, lb):
        cid = lax.axis_index("c")
        sid = lax.axis_index("s")
        wid = sid * _NC + cid
        base0 = wid * (_S_CHUNKS_PER_W * _S_CHUNK)

        @pl.when(sid == 0)
        def _():
            pltpu.sync_copy(zero_hbm, acc)
        plsc.subcore_barrier()

        pltpu.sync_copy(ridx_hbm.at[wid], iv)

        pltpu.async_copy(eout_hbm.at[pl.ds(base0, _S_CHUNK)], eva, la)
        pltpu.async_copy(eout_hbm.at[pl.ds(base0 + _S_CHUNK, _S_CHUNK)], evb, lb)

        def body(t, _):
            j0 = 2 * t
            j1 = j0 + 1
            pltpu.make_async_copy(
                eout_hbm.at[pl.ds(base0, _S_CHUNK)], eva, la).wait()
            pltpu.sync_copy(eva, acc.at[iv.at[j0]], add=True)

            @pl.when(j0 + 2 < _S_CHUNKS_PER_W)
            def _():
                pltpu.async_copy(
                    eout_hbm.at[pl.ds(base0 + (j0 + 2) * _S_CHUNK, _S_CHUNK)],
                    eva, la)

            pltpu.make_async_copy(
                eout_hbm.at[pl.ds(base0, _S_CHUNK)], evb, lb).wait()
            pltpu.sync_copy(evb, acc.at[iv.at[j1]], add=True)

            @pl.when(j1 + 2 < _S_CHUNKS_PER_W)
            def _():
                pltpu.async_copy(
                    eout_hbm.at[pl.ds(base0 + (j1 + 2) * _S_CHUNK, _S_CHUNK)],
                    evb, lb)

            return 0

        lax.fori_loop(0, _S_CHUNKS_PER_W // 2, body, 0)
        pltpu.make_async_copy(eout_hbm.at[pl.ds(base0, _S_CHUNK)], eva, la).wait()
        pltpu.sync_copy(eva, acc.at[iv.at[_S_CHUNKS_PER_W - 1]], add=True)

        plsc.subcore_barrier()

        @pl.when(sid == 0)
        def _():
            pltpu.sync_copy(acc, out_hbm.at[cid])

    return k(eout, ridx2, zeros_tab)


def _k5_body(nf_ref, agg_ref, w0a_ref, b0_ref, g_ref, bt_ref,
             w1_ref, b1_ref, out_ref):
    nf = nf_ref[...]
    h = _dot(nf, w0a_ref[...]) + agg_ref[0] + agg_ref[1] + b0_ref[...]
    mu = jnp.mean(h, axis=-1, keepdims=True)
    d = h - mu
    var = jnp.mean(d * d, axis=-1, keepdims=True)
    h = d / jnp.sqrt(var + 1e-5) * g_ref[...] + bt_ref[...]
    h = h * jax.nn.sigmoid(h)
    out_ref[...] = nf + _dot(h, w1_ref[...]) + b1_ref[...]


def _node_mlp(nf, agg2, w0a, b0, g, bt, w1, b1, interpret=False):
    blk = 2000
    grid = (N_NODES // blk,)
    full = lambda i: (0, 0)
    return pl.pallas_call(
        _k5_body,
        grid=grid,
        in_specs=[pl.BlockSpec((blk, NODE_DIM), lambda i: (i, 0)),
                  pl.BlockSpec((2, blk, HIDDEN), lambda i: (0, i, 0)),
                  pl.BlockSpec((NODE_DIM, HIDDEN), full),
                  pl.BlockSpec((1, HIDDEN), full),
                  pl.BlockSpec((1, HIDDEN), full),
                  pl.BlockSpec((1, HIDDEN), full),
                  pl.BlockSpec((HIDDEN, NODE_DIM), full),
                  pl.BlockSpec((1, NODE_DIM), full)],
        out_specs=pl.BlockSpec((blk, NODE_DIM), lambda i: (i, 0)),
        out_shape=jax.ShapeDtypeStruct((N_NODES, NODE_DIM), jnp.float32),
        interpret=interpret,
    )(nf, agg2, w0a, b0, g, bt, w1, b1)


def kernel(node_features, edge_features, edge_index,
           eW0, eb0, eg, ebt, eW1, eb1,
           nW0, nb0, ng, nbt, nW1, nb1):
    senders = edge_index[0]
    receivers = edge_index[1]

    w_sr = jnp.concatenate([eW0[:NODE_DIM], eW0[NODE_DIM:2 * NODE_DIM]], axis=1)
    nw0b = nW0[NODE_DIM:]
    we2 = jnp.concatenate([eW0[2 * NODE_DIM:], nw0b], axis=1)
    pad = jnp.zeros((_E_PAD - N_EDGES,), jnp.int32)
    idx_all = jnp.concatenate([senders, pad, receivers, pad]).reshape(
        2 * _G_NCHUNKS, _G_CHUNK)
    ridx2 = receivers.reshape(_NW, _S_CHUNKS_PER_W, _S_CHUNK)
    zeros_tab = jnp.zeros((N_NODES, HIDDEN), jnp.float32)
    row = lambda v: v.reshape(1, -1)

    ps, pr, w1cat, b1n = _node_project(node_features, w_sr, eW1, nw0b, row(eb1))
    hcat = _sc_gather(ps, pr, idx_all)
    edge_out, q = _edge_mlp(hcat, edge_features,
                            we2, row(eb0), row(eg), row(ebt), w1cat, row(eb1),
                            b1n)
    agg2 = _sc_scatter(q, ridx2, zeros_tab)
    node_out = _node_mlp(node_features, agg2,
                         nW0[:NODE_DIM], row(nb0), row(ng),
                         row(nbt), nW1, row(nb1))
    return (node_out, edge_out)


# speedup vs baseline: 1.3280x; 1.3280x over previous
import functools

import jax
import jax.numpy as jnp
from jax import lax
from jax.experimental import pallas as pl
from jax.experimental.pallas import tpu as pltpu
from jax.experimental.pallas import tpu_sc as plsc

N_NODES = 10000
N_EDGES = 320000
NODE_DIM = 128
EDGE_DIM = 16
HIDDEN = 128

_NC = 2
_NS = 16
_NW = _NC * _NS

_G_CHUNK = 128
_G_NCHUNKS = 2560
_E_PAD = _G_NCHUNKS * _G_CHUNK
_G_W0 = 96
_G_W1 = 64

_S_CHUNK = 80
_S_CHUNKS_PER_W = 125


def _dot(a, b, prec=jax.lax.Precision.HIGHEST):
    return jax.lax.dot_general(a, b, (((1,), (0,)), ((), ())),
                               precision=prec, preferred_element_type=jnp.float32)


def _k1_body(nf_ref, w_ref, w1_ref, nw0b_ref, eb1_ref, outs_ref, outr_ref,
             w1cat_ref, b1n_ref):
    p = _dot(nf_ref[...], w_ref[...])
    outs_ref[...] = p[:, :HIDDEN]
    outr_ref[...] = p[:, HIDDEN:]
    w1n = _dot(w1_ref[...], nw0b_ref[...])
    w1cat_ref[...] = jnp.concatenate([w1_ref[...], w1n], axis=1)
    b1n_ref[...] = _dot(eb1_ref[...], nw0b_ref[...])


def _node_project(nf, w_sr, w1, nw0b, eb1row, interpret=False):
    blk = 2000
    grid = (N_NODES // blk,)
    full = lambda i: (0, 0)
    return pl.pallas_call(
        _k1_body,
        grid=grid,
        in_specs=[pl.BlockSpec((blk, NODE_DIM), lambda i: (i, 0)),
                  pl.BlockSpec((NODE_DIM, 2 * HIDDEN), full),
                  pl.BlockSpec((HIDDEN, EDGE_DIM), full),
                  pl.BlockSpec((EDGE_DIM, HIDDEN), full),
                  pl.BlockSpec((1, EDGE_DIM), full)],
        out_specs=[pl.BlockSpec((blk, HIDDEN), lambda i: (i, 0)),
                   pl.BlockSpec((blk, HIDDEN), lambda i: (i, 0)),
                   pl.BlockSpec((HIDDEN, EDGE_DIM + HIDDEN), full),
                   pl.BlockSpec((1, HIDDEN), full)],
        out_shape=[jax.ShapeDtypeStruct((N_NODES, HIDDEN), jnp.float32),
                   jax.ShapeDtypeStruct((N_NODES, HIDDEN), jnp.float32),
                   jax.ShapeDtypeStruct((HIDDEN, EDGE_DIM + HIDDEN), jnp.float32),
                   jax.ShapeDtypeStruct((1, HIDDEN), jnp.float32)],
        interpret=interpret,
    )(nf, w_sr, w1, nw0b, eb1row)


def _sc_gather(ps, pr, sidx, ridx):
    mesh = plsc.VectorSubcoreMesh(core_axis_name="c", subcore_axis_name="s")

    @functools.partial(
        pl.kernel,
        out_type=jax.ShapeDtypeStruct((_E_PAD, HIDDEN), jnp.float32),
        mesh=mesh,
        scratch_types=[
            pltpu.VMEM((max(_G_W0, _G_W1), _G_CHUNK), jnp.int32),
            pltpu.VMEM((max(_G_W0, _G_W1), _G_CHUNK), jnp.int32),
            pltpu.VMEM((_G_CHUNK, HIDDEN), jnp.float32),
            pltpu.VMEM((_G_CHUNK, HIDDEN), jnp.float32),
            pltpu.VMEM((_G_CHUNK, HIDDEN), jnp.float32),
            pltpu.VMEM((_G_CHUNK, HIDDEN), jnp.float32),
            pltpu.VMEM((_G_CHUNK, HIDDEN), jnp.float32),
            pltpu.VMEM((_G_CHUNK, HIDDEN), jnp.float32),
            pltpu.SemaphoreType.DMA,
            pltpu.SemaphoreType.DMA,
            pltpu.SemaphoreType.DMA,
            pltpu.SemaphoreType.DMA,
            pltpu.SemaphoreType.DMA,
            pltpu.SemaphoreType.DMA,
        ],
    )
    def k(ps_hbm, pr_hbm, sidx_hbm, ridx_hbm, h_hbm,
          ivs, ivr, bsa, bra, bwa, bsb, brb, bwb,
          gsa, gra, gsb, grb, wa, wb):
        cid = lax.axis_index("c")
        sid = lax.axis_index("s")
        n_w = jnp.where(cid == 0, _G_W0, _G_W1)
        start = jnp.where(cid == 0, sid * _G_W0, 16 * _G_W0 + sid * _G_W1)

        @pl.when(cid == 0)
        def _():
            pltpu.sync_copy(sidx_hbm.at[pl.ds(sid * _G_W0, _G_W0)],
                            ivs.at[pl.ds(0, _G_W0)])
            pltpu.sync_copy(ridx_hbm.at[pl.ds(sid * _G_W0, _G_W0)],
                            ivr.at[pl.ds(0, _G_W0)])

        @pl.when(cid == 1)
        def _():
            base = 16 * _G_W0 + sid * _G_W1
            pltpu.sync_copy(sidx_hbm.at[pl.ds(base, _G_W1)],
                            ivs.at[pl.ds(0, _G_W1)])
            pltpu.sync_copy(ridx_hbm.at[pl.ds(base, _G_W1)],
                            ivr.at[pl.ds(0, _G_W1)])

        def fire(jl, bs, br, gs, gr):
            pltpu.async_copy(ps_hbm.at[ivs.at[jl]], bs, gs)
            pltpu.async_copy(pr_hbm.at[ivr.at[jl]], br, gr)

        def add_rows(bs, br, bw):
            def rb(i, _):
                for rr in range(4):
                    for c in range(0, HIDDEN, 16):
                        bw[4 * i + rr, pl.ds(c, 16)] = (
                            bs[4 * i + rr, pl.ds(c, 16)]
                            + br[4 * i + rr, pl.ds(c, 16)])
                return 0
            lax.fori_loop(0, _G_CHUNK // 4, rb, 0)

        fire(0, bsa, bra, gsa, gra)
        fire(1, bsb, brb, gsb, grb)

        def slot(jl, bs, br, bw, gs, gr, w, t):
            pltpu.make_async_copy(ps_hbm.at[ivs.at[jl]], bs, gs).wait()
            pltpu.make_async_copy(pr_hbm.at[ivr.at[jl]], br, gr).wait()

            @pl.when(t > 0)
            def _():
                pltpu.make_async_copy(bw, h_hbm.at[pl.ds(0, _G_CHUNK)],
                                      w).wait()
            add_rows(bs, br, bw)
            cg = start + jl
            pltpu.async_copy(bw, h_hbm.at[pl.ds(cg * _G_CHUNK, _G_CHUNK)], w)

            @pl.when(jl + 2 < n_w)
            def _():
                fire(jl + 2, bs, br, gs, gr)

        def body(t, _):
            slot(2 * t, bsa, bra, bwa, gsa, gra, wa, t)
            slot(2 * t + 1, bsb, brb, bwb, gsb, grb, wb, t)
            return 0

        lax.fori_loop(0, n_w // 2, body, 0)
        pltpu.make_async_copy(bwa, h_hbm.at[pl.ds(0, _G_CHUNK)], wa).wait()
        pltpu.make_async_copy(bwb, h_hbm.at[pl.ds(0, _G_CHUNK)], wb).wait()

    return k(ps, pr, sidx, ridx)


def _dot3(a, b):
    ah = a.astype(jnp.bfloat16)
    al = (a - ah.astype(jnp.float32)).astype(jnp.bfloat16)
    bh = b.astype(jnp.bfloat16)
    bl = (b - bh.astype(jnp.float32)).astype(jnp.bfloat16)
    d = lambda x, y: jax.lax.dot_general(
        x, y, (((1,), (0,)), ((), ())), preferred_element_type=jnp.float32)
    return d(ah, bh) + d(ah, bl) + d(al, bh)


def _dot1(a, b):
    return jax.lax.dot_general(
        a.astype(jnp.bfloat16), b.astype(jnp.bfloat16),
        (((1,), (0,)), ((), ())), preferred_element_type=jnp.float32)


def _k3_body(h_ref, ef_ref, we2_ref, b0_ref, g_ref, bt_ref,
             w1cat_ref, b1_ref, b1n_ref, out_ref, q_ref):
    e = ef_ref[...]
    s = _dot1(e, we2_ref[...])
    h = h_ref[...] + s[:, :HIDDEN] + b0_ref[...]
    mu = jnp.mean(h, axis=-1, keepdims=True)
    d = h - mu
    var = jnp.mean(d * d, axis=-1, keepdims=True)
    h = d / jnp.sqrt(var + 1e-5) * g_ref[...] + bt_ref[...]
    h = h * jax.nn.sigmoid(h)
    r = _dot3(h, w1cat_ref[...])
    out_ref[...] = e + r[:, :EDGE_DIM] + b1_ref[...]
    q_ref[...] = s[:, HIDDEN:] + r[:, EDGE_DIM:] + b1n_ref[...]


def _edge_mlp(h, ef, we2, b0, g, bt, w1cat, b1, b1n, interpret=False):
    blk = 2560
    grid = (N_EDGES // blk,)
    full = lambda i: (0, 0)
    return pl.pallas_call(
        _k3_body,
        grid=grid,
        in_specs=[pl.BlockSpec((blk, HIDDEN), lambda i: (i, 0)),
                  pl.BlockSpec((blk, EDGE_DIM), lambda i: (i, 0)),
                  pl.BlockSpec((EDGE_DIM, HIDDEN + HIDDEN), full),
                  pl.BlockSpec((1, HIDDEN), full),
                  pl.BlockSpec((1, HIDDEN), full),
                  pl.BlockSpec((1, HIDDEN), full),
                  pl.BlockSpec((HIDDEN, EDGE_DIM + HIDDEN), full),
                  pl.BlockSpec((1, EDGE_DIM), full),
                  pl.BlockSpec((1, HIDDEN), full)],
        out_specs=[pl.BlockSpec((blk, EDGE_DIM), lambda i: (i, 0)),
                   pl.BlockSpec((blk, HIDDEN), lambda i: (i, 0))],
        out_shape=[jax.ShapeDtypeStruct((N_EDGES, EDGE_DIM), jnp.float32),
                   jax.ShapeDtypeStruct((N_EDGES, HIDDEN), jnp.float32)],
        interpret=interpret,
    )(h, ef, we2, b0, g, bt, w1cat, b1, b1n)


def _sc_scatter(eout, ridx2, zeros_tab):
    mesh = plsc.VectorSubcoreMesh(core_axis_name="c", subcore_axis_name="s")

    @functools.partial(
        pl.kernel,
        out_type=jax.ShapeDtypeStruct((_NC, N_NODES, HIDDEN), jnp.float32),
        mesh=mesh,
        scratch_types=[
            pltpu.VMEM((_S_CHUNKS_PER_W, _S_CHUNK), jnp.int32),
            pltpu.VMEM((_S_CHUNK, HIDDEN), jnp.float32),
            pltpu.VMEM((_S_CHUNK, HIDDEN), jnp.float32),
            pltpu.VMEM_SHARED((N_NODES, HIDDEN), jnp.float32),
            pltpu.SemaphoreType.DMA,
            pltpu.SemaphoreType.DMA,
        ],
    )
    def k(eout_hbm, ridx_hbm, zero_hbm, out_hbm, iv, eva, evb, acc, la, lb):
        cid = lax.axis_index("c")
        sid = lax.axis_index("s")
        wid = sid * _NC + cid
        base0 = wid * (_S_CHUNKS_PER_W * _S_CHUNK)

        @pl.when(sid == 0)
        def _():
            pltpu.sync_copy(zero_hbm, acc)
        plsc.subcore_barrier()

        pltpu.sync_copy(ridx_hbm.at[wid], iv)

        pltpu.async_copy(eout_hbm.at[pl.ds(base0, _S_CHUNK)], eva, la)
        pltpu.async_copy(eout_hbm.at[pl.ds(base0 + _S_CHUNK, _S_CHUNK)], evb, lb)

        def body(t, _):
            j0 = 2 * t
            j1 = j0 + 1
            pltpu.make_async_copy(
                eout_hbm.at[pl.ds(base0, _S_CHUNK)], eva, la).wait()
            pltpu.sync_copy(eva, acc.at[iv.at[j0]], add=True)

            @pl.when(j0 + 2 < _S_CHUNKS_PER_W)
            def _():
                pltpu.async_copy(
                    eout_hbm.at[pl.ds(base0 + (j0 + 2) * _S_CHUNK, _S_CHUNK)],
                    eva, la)

            pltpu.make_async_copy(
                eout_hbm.at[pl.ds(base0, _S_CHUNK)], evb, lb).wait()
            pltpu.sync_copy(evb, acc.at[iv.at[j1]], add=True)

            @pl.when(j1 + 2 < _S_CHUNKS_PER_W)
            def _():
                pltpu.async_copy(
                    eout_hbm.at[pl.ds(base0 + (j1 + 2) * _S_CHUNK, _S_CHUNK)],
                    evb, lb)

            return 0

        lax.fori_loop(0, _S_CHUNKS_PER_W // 2, body, 0)
        pltpu.make_async_copy(eout_hbm.at[pl.ds(base0, _S_CHUNK)], eva, la).wait()
        pltpu.sync_copy(eva, acc.at[iv.at[_S_CHUNKS_PER_W - 1]], add=True)

        plsc.subcore_barrier()

        @pl.when(sid == 0)
        def _():
            pltpu.sync_copy(acc, out_hbm.at[cid])

    return k(eout, ridx2, zeros_tab)


def _k5_body(nf_ref, agg_ref, w0a_ref, b0_ref, g_ref, bt_ref,
             w1_ref, b1_ref, out_ref):
    nf = nf_ref[...]
    h = _dot(nf, w0a_ref[...]) + agg_ref[0] + agg_ref[1] + b0_ref[...]
    mu = jnp.mean(h, axis=-1, keepdims=True)
    d = h - mu
    var = jnp.mean(d * d, axis=-1, keepdims=True)
    h = d / jnp.sqrt(var + 1e-5) * g_ref[...] + bt_ref[...]
    h = h * jax.nn.sigmoid(h)
    out_ref[...] = nf + _dot(h, w1_ref[...]) + b1_ref[...]


def _node_mlp(nf, agg2, w0a, b0, g, bt, w1, b1, interpret=False):
    blk = 2000
    grid = (N_NODES // blk,)
    full = lambda i: (0, 0)
    return pl.pallas_call(
        _k5_body,
        grid=grid,
        in_specs=[pl.BlockSpec((blk, NODE_DIM), lambda i: (i, 0)),
                  pl.BlockSpec((2, blk, HIDDEN), lambda i: (0, i, 0)),
                  pl.BlockSpec((NODE_DIM, HIDDEN), full),
                  pl.BlockSpec((1, HIDDEN), full),
                  pl.BlockSpec((1, HIDDEN), full),
                  pl.BlockSpec((1, HIDDEN), full),
                  pl.BlockSpec((HIDDEN, NODE_DIM), full),
                  pl.BlockSpec((1, NODE_DIM), full)],
        out_specs=pl.BlockSpec((blk, NODE_DIM), lambda i: (i, 0)),
        out_shape=jax.ShapeDtypeStruct((N_NODES, NODE_DIM), jnp.float32),
        interpret=interpret,
    )(nf, agg2, w0a, b0, g, bt, w1, b1)


def kernel(node_features, edge_features, edge_index,
           eW0, eb0, eg, ebt, eW1, eb1,
           nW0, nb0, ng, nbt, nW1, nb1):
    senders = edge_index[0]
    receivers = edge_index[1]

    w_sr = jnp.concatenate([eW0[:NODE_DIM], eW0[NODE_DIM:2 * NODE_DIM]], axis=1)
    nw0b = nW0[NODE_DIM:]
    we2 = jnp.concatenate([eW0[2 * NODE_DIM:], nw0b], axis=1)
    pad = jnp.zeros((_E_PAD - N_EDGES,), jnp.int32)
    sidx = jnp.concatenate([senders, pad]).reshape(_G_NCHUNKS, _G_CHUNK)
    ridx = jnp.concatenate([receivers, pad]).reshape(_G_NCHUNKS, _G_CHUNK)
    ridx2 = receivers.reshape(_NW, _S_CHUNKS_PER_W, _S_CHUNK)
    zeros_tab = jnp.zeros((N_NODES, HIDDEN), jnp.float32)
    row = lambda v: v.reshape(1, -1)

    ps, pr, w1cat, b1n = _node_project(node_features, w_sr, eW1, nw0b, row(eb1))
    h = _sc_gather(ps, pr, sidx, ridx)
    edge_out, q = _edge_mlp(h, edge_features,
                            we2, row(eb0), row(eg), row(ebt), w1cat, row(eb1),
                            b1n)
    agg2 = _sc_scatter(q, ridx2, zeros_tab)
    node_out = _node_mlp(node_features, agg2,
                         nW0[:NODE_DIM], row(nb0), row(ng),
                         row(nbt), nW1, row(nb1))
    return (node_out, edge_out)


# speedup vs baseline: 1.4975x; 1.1276x over previous
import functools

import jax
import jax.numpy as jnp
from jax import lax
from jax.experimental import pallas as pl
from jax.experimental.pallas import tpu as pltpu
from jax.experimental.pallas import tpu_sc as plsc

N_NODES = 10000
N_EDGES = 320000
NODE_DIM = 128
EDGE_DIM = 16
HIDDEN = 128

_NC = 2
_NS = 16
_NW = _NC * _NS

_G_CHUNK = 128
_G_NCHUNKS = 2560
_E_PAD = _G_NCHUNKS * _G_CHUNK
_G_SEG = 1280
_G_W0 = 48
_G_W1 = 32

_S_CHUNK = 80
_S_CHUNKS_PER_W = 125


def _dot(a, b, prec=jax.lax.Precision.HIGHEST):
    return jax.lax.dot_general(a, b, (((1,), (0,)), ((), ())),
                               precision=prec, preferred_element_type=jnp.float32)


def _k1_body(nf_ref, w_ref, w1_ref, nw0b_ref, eb1_ref, outs_ref, outr_ref,
             w1cat_ref, b1n_ref):
    p = _dot(nf_ref[...], w_ref[...])
    outs_ref[...] = p[:, :HIDDEN]
    outr_ref[...] = p[:, HIDDEN:]
    w1n = _dot(w1_ref[...], nw0b_ref[...])
    w1cat_ref[...] = jnp.concatenate([w1_ref[...], w1n], axis=1)
    b1n_ref[...] = _dot(eb1_ref[...], nw0b_ref[...])


def _node_project(nf, w_sr, w1, nw0b, eb1row, interpret=False):
    blk = 2000
    grid = (N_NODES // blk,)
    full = lambda i: (0, 0)
    return pl.pallas_call(
        _k1_body,
        grid=grid,
        in_specs=[pl.BlockSpec((blk, NODE_DIM), lambda i: (i, 0)),
                  pl.BlockSpec((NODE_DIM, 2 * HIDDEN), full),
                  pl.BlockSpec((HIDDEN, EDGE_DIM), full),
                  pl.BlockSpec((EDGE_DIM, HIDDEN), full),
                  pl.BlockSpec((1, EDGE_DIM), full)],
        out_specs=[pl.BlockSpec((blk, HIDDEN), lambda i: (i, 0)),
                   pl.BlockSpec((blk, HIDDEN), lambda i: (i, 0)),
                   pl.BlockSpec((HIDDEN, EDGE_DIM + HIDDEN), full),
                   pl.BlockSpec((1, HIDDEN), full)],
        out_shape=[jax.ShapeDtypeStruct((N_NODES, HIDDEN), jnp.float32),
                   jax.ShapeDtypeStruct((N_NODES, HIDDEN), jnp.float32),
                   jax.ShapeDtypeStruct((HIDDEN, EDGE_DIM + HIDDEN), jnp.float32),
                   jax.ShapeDtypeStruct((1, HIDDEN), jnp.float32)],
        interpret=interpret,
    )(nf, w_sr, w1, nw0b, eb1row)


def _sc_gather(ps, pr, sidx, ridx):
    mesh = plsc.VectorSubcoreMesh(core_axis_name="c", subcore_axis_name="s")

    @functools.partial(
        pl.kernel,
        out_type=jax.ShapeDtypeStruct((_G_SEG * _G_CHUNK, HIDDEN), jnp.float32),
        mesh=mesh,
        scratch_types=[
            pltpu.VMEM((max(_G_W0, _G_W1), _G_CHUNK), jnp.int32),
            pltpu.VMEM((max(_G_W0, _G_W1), _G_CHUNK), jnp.int32),
            pltpu.VMEM((_G_CHUNK, HIDDEN), jnp.float32),
            pltpu.VMEM((_G_CHUNK, HIDDEN), jnp.float32),
            pltpu.VMEM((_G_CHUNK, HIDDEN), jnp.float32),
            pltpu.VMEM((_G_CHUNK, HIDDEN), jnp.float32),
            pltpu.VMEM((_G_CHUNK, HIDDEN), jnp.float32),
            pltpu.VMEM((_G_CHUNK, HIDDEN), jnp.float32),
            pltpu.SemaphoreType.DMA,
            pltpu.SemaphoreType.DMA,
            pltpu.SemaphoreType.DMA,
            pltpu.SemaphoreType.DMA,
            pltpu.SemaphoreType.DMA,
            pltpu.SemaphoreType.DMA,
        ],
    )
    def k(ps_hbm, pr_hbm, sidx_hbm, ridx_hbm, h_hbm,
          ivs, ivr, bsa, bra, bwa, bsb, brb, bwb,
          gsa, gra, gsb, grb, wa, wb):
        cid = lax.axis_index("c")
        sid = lax.axis_index("s")
        n_w = jnp.where(cid == 0, _G_W0, _G_W1)
        start = jnp.where(cid == 0, sid * _G_W0, 16 * _G_W0 + sid * _G_W1)

        @pl.when(cid == 0)
        def _():
            pltpu.sync_copy(sidx_hbm.at[pl.ds(sid * _G_W0, _G_W0)],
                            ivs.at[pl.ds(0, _G_W0)])
            pltpu.sync_copy(ridx_hbm.at[pl.ds(sid * _G_W0, _G_W0)],
                            ivr.at[pl.ds(0, _G_W0)])

        @pl.when(cid == 1)
        def _():
            base = 16 * _G_W0 + sid * _G_W1
            pltpu.sync_copy(sidx_hbm.at[pl.ds(base, _G_W1)],
                            ivs.at[pl.ds(0, _G_W1)])
            pltpu.sync_copy(ridx_hbm.at[pl.ds(base, _G_W1)],
                            ivr.at[pl.ds(0, _G_W1)])

        def fire(jl, bs, br, gs, gr):
            pltpu.async_copy(ps_hbm.at[ivs.at[jl]], bs, gs)
            pltpu.async_copy(pr_hbm.at[ivr.at[jl]], br, gr)

        def add_rows(bs, br, bw):
            def rb(i, _):
                for rr in range(4):
                    for c in range(0, HIDDEN, 16):
                        bw[4 * i + rr, pl.ds(c, 16)] = (
                            bs[4 * i + rr, pl.ds(c, 16)]
                            + br[4 * i + rr, pl.ds(c, 16)])
                return 0
            lax.fori_loop(0, _G_CHUNK // 4, rb, 0)

        fire(0, bsa, bra, gsa, gra)
        fire(1, bsb, brb, gsb, grb)

        def slot(jl, bs, br, bw, gs, gr, w, t):
            pltpu.make_async_copy(ps_hbm.at[ivs.at[jl]], bs, gs).wait()
            pltpu.make_async_copy(pr_hbm.at[ivr.at[jl]], br, gr).wait()

            @pl.when(t > 0)
            def _():
                pltpu.make_async_copy(bw, h_hbm.at[pl.ds(0, _G_CHUNK)],
                                      w).wait()
            add_rows(bs, br, bw)
            cg = start + jl
            pltpu.async_copy(bw, h_hbm.at[pl.ds(cg * _G_CHUNK, _G_CHUNK)], w)

            @pl.when(jl + 2 < n_w)
            def _():
                fire(jl + 2, bs, br, gs, gr)

        def body(t, _):
            slot(2 * t, bsa, bra, bwa, gsa, gra, wa, t)
            slot(2 * t + 1, bsb, brb, bwb, gsb, grb, wb, t)
            return 0

        lax.fori_loop(0, n_w // 2, body, 0)
        pltpu.make_async_copy(bwa, h_hbm.at[pl.ds(0, _G_CHUNK)], wa).wait()
        pltpu.make_async_copy(bwb, h_hbm.at[pl.ds(0, _G_CHUNK)], wb).wait()

    return k(ps, pr, sidx, ridx)


def _dot3(a, b):
    ah = a.astype(jnp.bfloat16)
    al = (a - ah.astype(jnp.float32)).astype(jnp.bfloat16)
    bh = b.astype(jnp.bfloat16)
    bl = (b - bh.astype(jnp.float32)).astype(jnp.bfloat16)
    d = lambda x, y: jax.lax.dot_general(
        x, y, (((1,), (0,)), ((), ())), preferred_element_type=jnp.float32)
    return d(ah, bh) + d(ah, bl) + d(al, bh)


def _dot1(a, b):
    return jax.lax.dot_general(
        a.astype(jnp.bfloat16), b.astype(jnp.bfloat16),
        (((1,), (0,)), ((), ())), preferred_element_type=jnp.float32)


def _k3_body(h_ref, ef_ref, we2_ref, b0_ref, g_ref, bt_ref,
             w1cat_ref, b1_ref, b1n_ref, out_ref, q_ref):
    e = ef_ref[...]
    s = _dot1(e, we2_ref[...])
    h = h_ref[...] + s[:, :HIDDEN] + b0_ref[...]
    mu = jnp.mean(h, axis=-1, keepdims=True)
    d = h - mu
    var = jnp.mean(d * d, axis=-1, keepdims=True)
    h = d / jnp.sqrt(var + 1e-5) * g_ref[...] + bt_ref[...]
    h = h * jax.nn.sigmoid(h)
    r = _dot3(h, w1cat_ref[...])
    out_ref[...] = e + r[:, :EDGE_DIM] + b1_ref[...]
    q_ref[...] = s[:, HIDDEN:] + r[:, EDGE_DIM:] + b1n_ref[...]


_K3_BLK = 2560
_K3_SEG_BLOCKS = (_G_SEG * _G_CHUNK) // _K3_BLK


def _edge_mlp(h, ef, we2, b0, g, bt, w1cat, b1, b1n, seg, eo_q=None,
              interpret=False):
    blk = _K3_BLK
    off = 0 if seg == 0 else _K3_SEG_BLOCKS
    n_blocks = (_K3_SEG_BLOCKS if seg == 0
                else N_EDGES // blk - _K3_SEG_BLOCKS)
    full = lambda i: (0, 0)
    seg_map = lambda i: (i + off, 0)
    in_specs = [pl.BlockSpec((blk, HIDDEN), lambda i: (i, 0)),
                pl.BlockSpec((blk, EDGE_DIM), seg_map),
                pl.BlockSpec((EDGE_DIM, HIDDEN + HIDDEN), full),
                pl.BlockSpec((1, HIDDEN), full),
                pl.BlockSpec((1, HIDDEN), full),
                pl.BlockSpec((1, HIDDEN), full),
                pl.BlockSpec((HIDDEN, EDGE_DIM + HIDDEN), full),
                pl.BlockSpec((1, EDGE_DIM), full),
                pl.BlockSpec((1, HIDDEN), full)]
    args = [h, ef, we2, b0, g, bt, w1cat, b1, b1n]
    aliases = {}
    body = _k3_body
    if seg == 1:
        in_specs += [pl.BlockSpec((8, EDGE_DIM), full),
                     pl.BlockSpec((8, HIDDEN), full)]
        args += list(eo_q)
        aliases = {9: 0, 10: 1}
        body = lambda *refs: _k3_body(*refs[:9], refs[11], refs[12])
    return pl.pallas_call(
        body,
        grid=(n_blocks,),
        in_specs=in_specs,
        out_specs=[pl.BlockSpec((blk, EDGE_DIM), seg_map),
                   pl.BlockSpec((blk, HIDDEN), seg_map)],
        out_shape=[jax.ShapeDtypeStruct((N_EDGES, EDGE_DIM), jnp.float32),
                   jax.ShapeDtypeStruct((N_EDGES, HIDDEN), jnp.float32)],
        input_output_aliases=aliases,
        interpret=interpret,
    )(*args)


def _sc_scatter(eout, ridx2, zeros_tab):
    mesh = plsc.VectorSubcoreMesh(core_axis_name="c", subcore_axis_name="s")

    @functools.partial(
        pl.kernel,
        out_type=jax.ShapeDtypeStruct((_NC, N_NODES, HIDDEN), jnp.float32),
        mesh=mesh,
        scratch_types=[
            pltpu.VMEM((_S_CHUNKS_PER_W, _S_CHUNK), jnp.int32),
            pltpu.VMEM((_S_CHUNK, HIDDEN), jnp.float32),
            pltpu.VMEM((_S_CHUNK, HIDDEN), jnp.float32),
            pltpu.VMEM_SHARED((N_NODES, HIDDEN), jnp.float32),
            pltpu.SemaphoreType.DMA,
            pltpu.SemaphoreType.DMA,
        ],
    )
    def k(eout_hbm, ridx_hbm, zero_hbm, out_hbm, iv, eva, evb, acc, la, lb):
        cid = lax.axis_index("c")
        sid = lax.axis_index("s")
        wid = sid * _NC + cid
        base0 = wid * (_S_CHUNKS_PER_W * _S_CHUNK)

        @pl.when(sid == 0)
        def _():
            pltpu.sync_copy(zero_hbm, acc)
        plsc.subcore_barrier()

        pltpu.sync_copy(ridx_hbm.at[wid], iv)

        pltpu.async_copy(eout_hbm.at[pl.ds(base0, _S_CHUNK)], eva, la)
        pltpu.async_copy(eout_hbm.at[pl.ds(base0 + _S_CHUNK, _S_CHUNK)], evb, lb)

        def body(t, _):
            j0 = 2 * t
            j1 = j0 + 1
            pltpu.make_async_copy(
                eout_hbm.at[pl.ds(base0, _S_CHUNK)], eva, la).wait()
            pltpu.sync_copy(eva, acc.at[iv.at[j0]], add=True)

            @pl.when(j0 + 2 < _S_CHUNKS_PER_W)
            def _():
                pltpu.async_copy(
                    eout_hbm.at[pl.ds(base0 + (j0 + 2) * _S_CHUNK, _S_CHUNK)],
                    eva, la)

            pltpu.make_async_copy(
                eout_hbm.at[pl.ds(base0, _S_CHUNK)], evb, lb).wait()
            pltpu.sync_copy(evb, acc.at[iv.at[j1]], add=True)

            @pl.when(j1 + 2 < _S_CHUNKS_PER_W)
            def _():
                pltpu.async_copy(
                    eout_hbm.at[pl.ds(base0 + (j1 + 2) * _S_CHUNK, _S_CHUNK)],
                    evb, lb)

            return 0

        lax.fori_loop(0, _S_CHUNKS_PER_W // 2, body, 0)
        pltpu.make_async_copy(eout_hbm.at[pl.ds(base0, _S_CHUNK)], eva, la).wait()
        pltpu.sync_copy(eva, acc.at[iv.at[_S_CHUNKS_PER_W - 1]], add=True)

        plsc.subcore_barrier()

        @pl.when(sid == 0)
        def _():
            pltpu.sync_copy(acc, out_hbm.at[cid])

    return k(eout, ridx2, zeros_tab)


def _k5_body(nf_ref, agg_ref, w0a_ref, b0_ref, g_ref, bt_ref,
             w1_ref, b1_ref, out_ref):
    nf = nf_ref[...]
    h = _dot(nf, w0a_ref[...]) + agg_ref[0] + agg_ref[1] + b0_ref[...]
    mu = jnp.mean(h, axis=-1, keepdims=True)
    d = h - mu
    var = jnp.mean(d * d, axis=-1, keepdims=True)
    h = d / jnp.sqrt(var + 1e-5) * g_ref[...] + bt_ref[...]
    h = h * jax.nn.sigmoid(h)
    out_ref[...] = nf + _dot(h, w1_ref[...]) + b1_ref[...]


def _node_mlp(nf, agg2, w0a, b0, g, bt, w1, b1, interpret=False):
    blk = 2000
    grid = (N_NODES // blk,)
    full = lambda i: (0, 0)
    return pl.pallas_call(
        _k5_body,
        grid=grid,
        in_specs=[pl.BlockSpec((blk, NODE_DIM), lambda i: (i, 0)),
                  pl.BlockSpec((2, blk, HIDDEN), lambda i: (0, i, 0)),
                  pl.BlockSpec((NODE_DIM, HIDDEN), full),
                  pl.BlockSpec((1, HIDDEN), full),
                  pl.BlockSpec((1, HIDDEN), full),
                  pl.BlockSpec((1, HIDDEN), full),
                  pl.BlockSpec((HIDDEN, NODE_DIM), full),
                  pl.BlockSpec((1, NODE_DIM), full)],
        out_specs=pl.BlockSpec((blk, NODE_DIM), lambda i: (i, 0)),
        out_shape=jax.ShapeDtypeStruct((N_NODES, NODE_DIM), jnp.float32),
        interpret=interpret,
    )(nf, agg2, w0a, b0, g, bt, w1, b1)


def kernel(node_features, edge_features, edge_index,
           eW0, eb0, eg, ebt, eW1, eb1,
           nW0, nb0, ng, nbt, nW1, nb1):
    senders = edge_index[0]
    receivers = edge_index[1]

    w_sr = jnp.concatenate([eW0[:NODE_DIM], eW0[NODE_DIM:2 * NODE_DIM]], axis=1)
    nw0b = nW0[NODE_DIM:]
    we2 = jnp.concatenate([eW0[2 * NODE_DIM:], nw0b], axis=1)
    pad = jnp.zeros((_E_PAD - N_EDGES,), jnp.int32)
    sidx = jnp.concatenate([senders, pad]).reshape(_G_NCHUNKS, _G_CHUNK)
    ridx = jnp.concatenate([receivers, pad]).reshape(_G_NCHUNKS, _G_CHUNK)
    ridx2 = receivers.reshape(_NW, _S_CHUNKS_PER_W, _S_CHUNK)
    zeros_tab = jnp.zeros((N_NODES, HIDDEN), jnp.float32)
    row = lambda v: v.reshape(1, -1)

    ps, pr, w1cat, b1n = _node_project(node_features, w_sr, eW1, nw0b, row(eb1))
    h0 = _sc_gather(ps, pr, sidx[:_G_SEG], ridx[:_G_SEG])
    h1 = _sc_gather(ps, pr, sidx[_G_SEG:], ridx[_G_SEG:])
    eo_q0 = _edge_mlp(h0, edge_features, we2, row(eb0), row(eg), row(ebt),
                      w1cat, row(eb1), b1n, seg=0)
    edge_out, q = _edge_mlp(h1, edge_features, we2, row(eb0), row(eg),
                            row(ebt), w1cat, row(eb1), b1n, seg=1,
                            eo_q=eo_q0)
    agg2 = _sc_scatter(q, ridx2, zeros_tab)
    node_out = _node_mlp(node_features, agg2,
                         nW0[:NODE_DIM], row(nb0), row(ng),
                         row(nbt), nW1, row(nb1))
    return (node_out, edge_out)


# speedup vs baseline: 1.6159x; 1.0791x over previous
import functools

import jax
import jax.numpy as jnp
from jax import lax
from jax.experimental import pallas as pl
from jax.experimental.pallas import tpu as pltpu
from jax.experimental.pallas import tpu_sc as plsc

N_NODES = 10000
N_EDGES = 320000
NODE_DIM = 128
EDGE_DIM = 16
HIDDEN = 128

_NC = 2
_NS = 16
_NW = _NC * _NS

_G_CHUNK = 128
_G_NCHUNKS = 2560
_E_PAD = _G_NCHUNKS * _G_CHUNK
_G_NSEG = 4
_G_SEG = 640
_G_W0 = 24
_G_W1 = 16

_S_CHUNK = 80
_S_CHUNKS_PER_W = 125


def _dot(a, b, prec=jax.lax.Precision.HIGHEST):
    return jax.lax.dot_general(a, b, (((1,), (0,)), ((), ())),
                               precision=prec, preferred_element_type=jnp.float32)


def _k1_body(nf_ref, w_ref, w1_ref, nw0b_ref, eb1_ref, outs_ref, outr_ref,
             w1cat_ref, b1n_ref):
    p = _dot(nf_ref[...], w_ref[...])
    outs_ref[...] = p[:, :HIDDEN]
    outr_ref[...] = p[:, HIDDEN:]
    w1n = _dot(w1_ref[...], nw0b_ref[...])
    w1cat_ref[...] = jnp.concatenate([w1_ref[...], w1n], axis=1)
    b1n_ref[...] = _dot(eb1_ref[...], nw0b_ref[...])


def _node_project(nf, w_sr, w1, nw0b, eb1row, interpret=False):
    blk = 2000
    grid = (N_NODES // blk,)
    full = lambda i: (0, 0)
    return pl.pallas_call(
        _k1_body,
        grid=grid,
        in_specs=[pl.BlockSpec((blk, NODE_DIM), lambda i: (i, 0)),
                  pl.BlockSpec((NODE_DIM, 2 * HIDDEN), full),
                  pl.BlockSpec((HIDDEN, EDGE_DIM), full),
                  pl.BlockSpec((EDGE_DIM, HIDDEN), full),
                  pl.BlockSpec((1, EDGE_DIM), full)],
        out_specs=[pl.BlockSpec((blk, HIDDEN), lambda i: (i, 0)),
                   pl.BlockSpec((blk, HIDDEN), lambda i: (i, 0)),
                   pl.BlockSpec((HIDDEN, EDGE_DIM + HIDDEN), full),
                   pl.BlockSpec((1, HIDDEN), full)],
        out_shape=[jax.ShapeDtypeStruct((N_NODES, HIDDEN), jnp.float32),
                   jax.ShapeDtypeStruct((N_NODES, HIDDEN), jnp.float32),
                   jax.ShapeDtypeStruct((HIDDEN, EDGE_DIM + HIDDEN), jnp.float32),
                   jax.ShapeDtypeStruct((1, HIDDEN), jnp.float32)],
        interpret=interpret,
    )(nf, w_sr, w1, nw0b, eb1row)


def _sc_gather(ps, pr, sidx, ridx):
    mesh = plsc.VectorSubcoreMesh(core_axis_name="c", subcore_axis_name="s")

    @functools.partial(
        pl.kernel,
        out_type=jax.ShapeDtypeStruct((_G_SEG * _G_CHUNK, HIDDEN), jnp.float32),
        mesh=mesh,
        scratch_types=[
            pltpu.VMEM((max(_G_W0, _G_W1), _G_CHUNK), jnp.int32),
            pltpu.VMEM((max(_G_W0, _G_W1), _G_CHUNK), jnp.int32),
            pltpu.VMEM((_G_CHUNK, HIDDEN), jnp.float32),
            pltpu.VMEM((_G_CHUNK, HIDDEN), jnp.float32),
            pltpu.VMEM((_G_CHUNK, HIDDEN), jnp.float32),
            pltpu.VMEM((_G_CHUNK, HIDDEN), jnp.float32),
            pltpu.VMEM((_G_CHUNK, HIDDEN), jnp.float32),
            pltpu.VMEM((_G_CHUNK, HIDDEN), jnp.float32),
            pltpu.SemaphoreType.DMA,
            pltpu.SemaphoreType.DMA,
            pltpu.SemaphoreType.DMA,
            pltpu.SemaphoreType.DMA,
            pltpu.SemaphoreType.DMA,
            pltpu.SemaphoreType.DMA,
        ],
    )
    def k(ps_hbm, pr_hbm, sidx_hbm, ridx_hbm, h_hbm,
          ivs, ivr, bsa, bra, bwa, bsb, brb, bwb,
          gsa, gra, gsb, grb, wa, wb):
        cid = lax.axis_index("c")
        sid = lax.axis_index("s")
        n_w = jnp.where(cid == 0, _G_W0, _G_W1)
        start = jnp.where(cid == 0, sid * _G_W0, 16 * _G_W0 + sid * _G_W1)

        @pl.when(cid == 0)
        def _():
            pltpu.sync_copy(sidx_hbm.at[pl.ds(sid * _G_W0, _G_W0)],
                            ivs.at[pl.ds(0, _G_W0)])
            pltpu.sync_copy(ridx_hbm.at[pl.ds(sid * _G_W0, _G_W0)],
                            ivr.at[pl.ds(0, _G_W0)])

        @pl.when(cid == 1)
        def _():
            base = 16 * _G_W0 + sid * _G_W1
            pltpu.sync_copy(sidx_hbm.at[pl.ds(base, _G_W1)],
                            ivs.at[pl.ds(0, _G_W1)])
            pltpu.sync_copy(ridx_hbm.at[pl.ds(base, _G_W1)],
                            ivr.at[pl.ds(0, _G_W1)])

        def fire(jl, bs, br, gs, gr):
            pltpu.async_copy(ps_hbm.at[ivs.at[jl]], bs, gs)
            pltpu.async_copy(pr_hbm.at[ivr.at[jl]], br, gr)

        def add_rows(bs, br, bw):
            def rb(i, _):
                for rr in range(4):
                    for c in range(0, HIDDEN, 16):
                        bw[4 * i + rr, pl.ds(c, 16)] = (
                            bs[4 * i + rr, pl.ds(c, 16)]
                            + br[4 * i + rr, pl.ds(c, 16)])
                return 0
            lax.fori_loop(0, _G_CHUNK // 4, rb, 0)

        fire(0, bsa, bra, gsa, gra)
        fire(1, bsb, brb, gsb, grb)

        def slot(jl, bs, br, bw, gs, gr, w, t):
            pltpu.make_async_copy(ps_hbm.at[ivs.at[jl]], bs, gs).wait()
            pltpu.make_async_copy(pr_hbm.at[ivr.at[jl]], br, gr).wait()

            @pl.when(t > 0)
            def _():
                pltpu.make_async_copy(bw, h_hbm.at[pl.ds(0, _G_CHUNK)],
                                      w).wait()
            add_rows(bs, br, bw)
            cg = start + jl
            pltpu.async_copy(bw, h_hbm.at[pl.ds(cg * _G_CHUNK, _G_CHUNK)], w)

            @pl.when(jl + 2 < n_w)
            def _():
                fire(jl + 2, bs, br, gs, gr)

        def body(t, _):
            slot(2 * t, bsa, bra, bwa, gsa, gra, wa, t)
            slot(2 * t + 1, bsb, brb, bwb, gsb, grb, wb, t)
            return 0

        lax.fori_loop(0, n_w // 2, body, 0)
        pltpu.make_async_copy(bwa, h_hbm.at[pl.ds(0, _G_CHUNK)], wa).wait()
        pltpu.make_async_copy(bwb, h_hbm.at[pl.ds(0, _G_CHUNK)], wb).wait()

    return k(ps, pr, sidx, ridx)


def _dot3(a, b):
    ah = a.astype(jnp.bfloat16)
    al = (a - ah.astype(jnp.float32)).astype(jnp.bfloat16)
    bh = b.astype(jnp.bfloat16)
    bl = (b - bh.astype(jnp.float32)).astype(jnp.bfloat16)
    d = lambda x, y: jax.lax.dot_general(
        x, y, (((1,), (0,)), ((), ())), preferred_element_type=jnp.float32)
    return d(ah, bh) + d(ah, bl) + d(al, bh)


def _dot1(a, b):
    return jax.lax.dot_general(
        a.astype(jnp.bfloat16), b.astype(jnp.bfloat16),
        (((1,), (0,)), ((), ())), preferred_element_type=jnp.float32)


def _k3_body(h_ref, ef_ref, we2_ref, b0_ref, g_ref, bt_ref,
             w1cat_ref, b1_ref, b1n_ref, out_ref, q_ref):
    e = ef_ref[...]
    s = _dot1(e, we2_ref[...])
    h = h_ref[...] + s[:, :HIDDEN] + b0_ref[...]
    mu = jnp.mean(h, axis=-1, keepdims=True)
    d = h - mu
    var = jnp.mean(d * d, axis=-1, keepdims=True)
    h = d / jnp.sqrt(var + 1e-5) * g_ref[...] + bt_ref[...]
    h = h * jax.nn.sigmoid(h)
    r = _dot3(h, w1cat_ref[...])
    out_ref[...] = e + r[:, :EDGE_DIM] + b1_ref[...]
    q_ref[...] = s[:, HIDDEN:] + r[:, EDGE_DIM:] + b1n_ref[...]


_K3_BLK = 2560
_K3_SEG_BLOCKS = (_G_SEG * _G_CHUNK) // _K3_BLK


def _edge_mlp(h, ef, we2, b0, g, bt, w1cat, b1, b1n, seg, eo_q=None,
              interpret=False):
    blk = _K3_BLK
    off = seg * _K3_SEG_BLOCKS
    n_blocks = (_K3_SEG_BLOCKS if seg < _G_NSEG - 1
                else N_EDGES // blk - off)
    full = lambda i: (0, 0)
    seg_map = lambda i: (i + off, 0)
    in_specs = [pl.BlockSpec((blk, HIDDEN), lambda i: (i, 0)),
                pl.BlockSpec((blk, EDGE_DIM), seg_map),
                pl.BlockSpec((EDGE_DIM, HIDDEN + HIDDEN), full),
                pl.BlockSpec((1, HIDDEN), full),
                pl.BlockSpec((1, HIDDEN), full),
                pl.BlockSpec((1, HIDDEN), full),
                pl.BlockSpec((HIDDEN, EDGE_DIM + HIDDEN), full),
                pl.BlockSpec((1, EDGE_DIM), full),
                pl.BlockSpec((1, HIDDEN), full)]
    args = [h, ef, we2, b0, g, bt, w1cat, b1, b1n]
    aliases = {}
    body = _k3_body
    if seg > 0:
        in_specs += [pl.BlockSpec((8, EDGE_DIM), full),
                     pl.BlockSpec((8, HIDDEN), full)]
        args += list(eo_q)
        aliases = {9: 0, 10: 1}
        body = lambda *refs: _k3_body(*refs[:9], refs[11], refs[12])
    return pl.pallas_call(
        body,
        grid=(n_blocks,),
        in_specs=in_specs,
        out_specs=[pl.BlockSpec((blk, EDGE_DIM), seg_map),
                   pl.BlockSpec((blk, HIDDEN), seg_map)],
        out_shape=[jax.ShapeDtypeStruct((N_EDGES, EDGE_DIM), jnp.float32),
                   jax.ShapeDtypeStruct((N_EDGES, HIDDEN), jnp.float32)],
        input_output_aliases=aliases,
        interpret=interpret,
    )(*args)


def _sc_scatter(eout, ridx2, zeros_tab):
    mesh = plsc.VectorSubcoreMesh(core_axis_name="c", subcore_axis_name="s")

    @functools.partial(
        pl.kernel,
        out_type=jax.ShapeDtypeStruct((_NC, N_NODES, HIDDEN), jnp.float32),
        mesh=mesh,
        scratch_types=[
            pltpu.VMEM((_S_CHUNKS_PER_W, _S_CHUNK), jnp.int32),
            pltpu.VMEM((_S_CHUNK, HIDDEN), jnp.float32),
            pltpu.VMEM((_S_CHUNK, HIDDEN), jnp.float32),
            pltpu.VMEM_SHARED((N_NODES, HIDDEN), jnp.float32),
            pltpu.SemaphoreType.DMA,
            pltpu.SemaphoreType.DMA,
        ],
    )
    def k(eout_hbm, ridx_hbm, zero_hbm, out_hbm, iv, eva, evb, acc, la, lb):
        cid = lax.axis_index("c")
        sid = lax.axis_index("s")
        wid = sid * _NC + cid
        base0 = wid * (_S_CHUNKS_PER_W * _S_CHUNK)

        @pl.when(sid == 0)
        def _():
            pltpu.sync_copy(zero_hbm, acc)
        plsc.subcore_barrier()

        pltpu.sync_copy(ridx_hbm.at[wid], iv)

        pltpu.async_copy(eout_hbm.at[pl.ds(base0, _S_CHUNK)], eva, la)
        pltpu.async_copy(eout_hbm.at[pl.ds(base0 + _S_CHUNK, _S_CHUNK)], evb, lb)

        def body(t, _):
            j0 = 2 * t
            j1 = j0 + 1
            pltpu.make_async_copy(
                eout_hbm.at[pl.ds(base0, _S_CHUNK)], eva, la).wait()
            pltpu.sync_copy(eva, acc.at[iv.at[j0]], add=True)

            @pl.when(j0 + 2 < _S_CHUNKS_PER_W)
            def _():
                pltpu.async_copy(
                    eout_hbm.at[pl.ds(base0 + (j0 + 2) * _S_CHUNK, _S_CHUNK)],
                    eva, la)

            pltpu.make_async_copy(
                eout_hbm.at[pl.ds(base0, _S_CHUNK)], evb, lb).wait()
            pltpu.sync_copy(evb, acc.at[iv.at[j1]], add=True)

            @pl.when(j1 + 2 < _S_CHUNKS_PER_W)
            def _():
                pltpu.async_copy(
                    eout_hbm.at[pl.ds(base0 + (j1 + 2) * _S_CHUNK, _S_CHUNK)],
                    evb, lb)

            return 0

        lax.fori_loop(0, _S_CHUNKS_PER_W // 2, body, 0)
        pltpu.make_async_copy(eout_hbm.at[pl.ds(base0, _S_CHUNK)], eva, la).wait()
        pltpu.sync_copy(eva, acc.at[iv.at[_S_CHUNKS_PER_W - 1]], add=True)

        plsc.subcore_barrier()

        @pl.when(sid == 0)
        def _():
            pltpu.sync_copy(acc, out_hbm.at[cid])

    return k(eout, ridx2, zeros_tab)


def _k5_body(nf_ref, agg_ref, w0a_ref, b0_ref, g_ref, bt_ref,
             w1_ref, b1_ref, out_ref):
    nf = nf_ref[...]
    h = _dot(nf, w0a_ref[...]) + agg_ref[0] + agg_ref[1] + b0_ref[...]
    mu = jnp.mean(h, axis=-1, keepdims=True)
    d = h - mu
    var = jnp.mean(d * d, axis=-1, keepdims=True)
    h = d / jnp.sqrt(var + 1e-5) * g_ref[...] + bt_ref[...]
    h = h * jax.nn.sigmoid(h)
    out_ref[...] = nf + _dot(h, w1_ref[...]) + b1_ref[...]


def _node_mlp(nf, agg2, w0a, b0, g, bt, w1, b1, interpret=False):
    blk = 2000
    grid = (N_NODES // blk,)
    full = lambda i: (0, 0)
    return pl.pallas_call(
        _k5_body,
        grid=grid,
        in_specs=[pl.BlockSpec((blk, NODE_DIM), lambda i: (i, 0)),
                  pl.BlockSpec((2, blk, HIDDEN), lambda i: (0, i, 0)),
                  pl.BlockSpec((NODE_DIM, HIDDEN), full),
                  pl.BlockSpec((1, HIDDEN), full),
                  pl.BlockSpec((1, HIDDEN), full),
                  pl.BlockSpec((1, HIDDEN), full),
                  pl.BlockSpec((HIDDEN, NODE_DIM), full),
                  pl.BlockSpec((1, NODE_DIM), full)],
        out_specs=pl.BlockSpec((blk, NODE_DIM), lambda i: (i, 0)),
        out_shape=jax.ShapeDtypeStruct((N_NODES, NODE_DIM), jnp.float32),
        interpret=interpret,
    )(nf, agg2, w0a, b0, g, bt, w1, b1)


def kernel(node_features, edge_features, edge_index,
           eW0, eb0, eg, ebt, eW1, eb1,
           nW0, nb0, ng, nbt, nW1, nb1):
    senders = edge_index[0]
    receivers = edge_index[1]

    w_sr = jnp.concatenate([eW0[:NODE_DIM], eW0[NODE_DIM:2 * NODE_DIM]], axis=1)
    nw0b = nW0[NODE_DIM:]
    we2 = jnp.concatenate([eW0[2 * NODE_DIM:], nw0b], axis=1)
    pad = jnp.zeros((_E_PAD - N_EDGES,), jnp.int32)
    sidx = jnp.concatenate([senders, pad]).reshape(_G_NCHUNKS, _G_CHUNK)
    ridx = jnp.concatenate([receivers, pad]).reshape(_G_NCHUNKS, _G_CHUNK)
    ridx2 = receivers.reshape(_NW, _S_CHUNKS_PER_W, _S_CHUNK)
    zeros_tab = jnp.zeros((N_NODES, HIDDEN), jnp.float32)
    row = lambda v: v.reshape(1, -1)

    ps, pr, w1cat, b1n = _node_project(node_features, w_sr, eW1, nw0b, row(eb1))
    eo_q = None
    for seg in range(_G_NSEG):
        h_seg = _sc_gather(ps, pr, sidx[seg * _G_SEG:(seg + 1) * _G_SEG],
                           ridx[seg * _G_SEG:(seg + 1) * _G_SEG])
        eo_q = _edge_mlp(h_seg, edge_features, we2, row(eb0), row(eg),
                         row(ebt), w1cat, row(eb1), b1n, seg=seg, eo_q=eo_q)
    edge_out, q = eo_q
    agg2 = _sc_scatter(q, ridx2, zeros_tab)
    node_out = _node_mlp(node_features, agg2,
                         nW0[:NODE_DIM], row(nb0), row(ng),
                         row(nbt), nW1, row(nb1))
    return (node_out, edge_out)
